# initial kernel scaffold (unmeasured)
import jax
import jax.numpy as jnp
from jax import lax
from jax.experimental import pallas as pl
from jax.experimental.pallas import tpu as pltpu

N_DEV = 4
B = 2
SQ = 512
SKV_LOC = 512
HL = 8
DH = 64
DM = 768
WIN = 128
KV_USED = SQ + WIN
KV1 = KV_USED - SKV_LOC
BLK = SQ // N_DEV
SCALE = 0.125
NEG = -1e9


def _mrc(src, dst, ssem, rsem, dev):
    return pltpu.make_async_remote_copy(
        src_ref=src, dst_ref=dst, send_sem=ssem, recv_sem=rsem,
        device_id=(dev,), device_id_type=pl.DeviceIdType.MESH,
    )


def kernel(x, Wq, K_ext, V_ext, Wo):
    def body(x_ref, wq_ref, k_ref, v_ref, wo_ref, out_ref,
             kbuf, vbuf, psum, arbuf,
             ksend, vsend, p1recv, rssend, rsrecv, agsend, agrecv):
        my = lax.axis_index("i")

        bar = pltpu.get_barrier_semaphore()
        for j in range(N_DEV):
            @pl.when(my != j)
            def _(j=j):
                pl.semaphore_signal(bar, inc=1, device_id=(j,),
                                    device_id_type=pl.DeviceIdType.MESH)
        pl.semaphore_wait(bar, N_DEV - 1)

        for s, (dst_off, n_rows) in ((0, (0, SKV_LOC)), (1, (SKV_LOC, KV1))):
            @pl.when(my == s)
            def _(s=s, dst_off=dst_off, n_rows=n_rows):
                descs = []
                for j in range(N_DEV):
                    if j == s:
                        continue
                    dk = _mrc(k_ref.at[:, pl.ds(0, n_rows), pl.ds(HL * j, HL), :],
                              kbuf.at[:, pl.ds(dst_off, n_rows)],
                              ksend.at[j], p1recv.at[2 * s], j)
                    dv = _mrc(v_ref.at[:, pl.ds(0, n_rows), pl.ds(HL * j, HL), :],
                              vbuf.at[:, pl.ds(dst_off, n_rows)],
                              vsend.at[j], p1recv.at[2 * s + 1], j)
                    dk.start()
                    dv.start()
                    descs.append((dk, dv))
                kbuf[:, dst_off:dst_off + n_rows] = \
                    k_ref[:, 0:n_rows, HL * s:HL * (s + 1), :]
                vbuf[:, dst_off:dst_off + n_rows] = \
                    v_ref[:, 0:n_rows, HL * s:HL * (s + 1), :]
                for dk, dv in descs:
                    dk.wait_send()
                    dv.wait_send()

        for s, (dst_off, n_rows) in ((0, (0, SKV_LOC)), (1, (SKV_LOC, KV1))):
            @pl.when(my != s)
            def _(s=s, dst_off=dst_off, n_rows=n_rows):
                rk = _mrc(kbuf.at[:, pl.ds(dst_off, n_rows)],
                          kbuf.at[:, pl.ds(dst_off, n_rows)],
                          ksend.at[s], p1recv.at[2 * s], my)
                rv = _mrc(vbuf.at[:, pl.ds(dst_off, n_rows)],
                          vbuf.at[:, pl.ds(dst_off, n_rows)],
                          vsend.at[s], p1recv.at[2 * s + 1], my)
                rk.wait_recv()
                rv.wait_recv()

        xq = lax.dot_general(
            jnp.reshape(x_ref[...], (B * SQ, DM)), wq_ref[...],
            (((1,), (0,)), ((), ())), preferred_element_type=jnp.float32)

        kv_k = kbuf[...]
        kv_v = vbuf[...]
        wo = wo_ref[...]

        qi = lax.broadcasted_iota(jnp.int32, (SQ, KV_USED), 0)
        ki = lax.broadcasted_iota(jnp.int32, (SQ, KV_USED), 1)
        mask = jnp.abs(qi - ki) <= WIN

        for b in range(B):
            ctxs = []
            for h in range(HL):
                q_bh = xq[b * SQ:(b + 1) * SQ, h * DH:(h + 1) * DH]
                k_bh = kv_k[b, :, h, :]
                v_bh = kv_v[b, :, h, :]
                sc = lax.dot_general(
                    q_bh, k_bh, (((1,), (1,)), ((), ())),
                    preferred_element_type=jnp.float32) * SCALE
                sc = jnp.where(mask, sc, NEG)
                m = jnp.max(sc, axis=-1, keepdims=True)
                w = jnp.exp(sc - m)
                w = w / jnp.sum(w, axis=-1, keepdims=True)
                ctxs.append(lax.dot_general(
                    w, v_bh, (((1,), (0,)), ((), ())),
                    preferred_element_type=jnp.float32))
            ctx_b = jnp.concatenate(ctxs, axis=1)
            psum[b] = lax.dot_general(
                ctx_b, wo, (((1,), (0,)), ((), ())),
                preferred_element_type=jnp.float32)

        for s in range(N_DEV):
            @pl.when(my == s)
            def _(s=s):
                for j in range(N_DEV):
                    if j == s:
                        continue
                    _mrc(psum.at[:, pl.ds(j * BLK, BLK), :], arbuf.at[s],
                         rssend.at[j], rsrecv.at[s], j).start()
                arbuf[s] = psum[:, s * BLK:(s + 1) * BLK, :]

        for s in range(N_DEV):
            @pl.when(my != s)
            def _(s=s):
                _mrc(arbuf.at[s], arbuf.at[s],
                     rssend.at[s], rsrecv.at[s], my).wait_recv()

        red = arbuf[0] + arbuf[1] + arbuf[2] + arbuf[3]

        for s in range(N_DEV):
            @pl.when(my == s)
            def _(s=s):
                out_ref[:, s * BLK:(s + 1) * BLK, :] = red
                for j in range(N_DEV):
                    if j == s:
                        continue
                    _mrc(out_ref.at[:, pl.ds(s * BLK, BLK), :],
                         out_ref.at[:, pl.ds(s * BLK, BLK), :],
                         agsend.at[j], agrecv.at[s], j).start()

        for s in range(N_DEV):
            @pl.when(my != s)
            def _(s=s):
                _mrc(out_ref.at[:, pl.ds(s * BLK, BLK), :],
                     out_ref.at[:, pl.ds(s * BLK, BLK), :],
                     agsend.at[s], agrecv.at[s], my).wait_recv()

        for s in range(N_DEV):
            @pl.when(my == s)
            def _(s=s):
                for j in range(N_DEV):
                    if j == s:
                        continue
                    _mrc(psum.at[:, pl.ds(j * BLK, BLK), :], arbuf.at[s],
                         rssend.at[j], rsrecv.at[s], j).wait_send()
                    _mrc(out_ref.at[:, pl.ds(s * BLK, BLK), :],
                         out_ref.at[:, pl.ds(s * BLK, BLK), :],
                         agsend.at[j], agrecv.at[s], j).wait_send()

    return pl.pallas_call(
        body,
        out_shape=jax.ShapeDtypeStruct((B, SQ, DM), jnp.float32),
        in_specs=[pl.BlockSpec(memory_space=pltpu.VMEM)] * 5,
        out_specs=pl.BlockSpec(memory_space=pltpu.VMEM),
        scratch_shapes=[
            pltpu.VMEM((B, KV_USED, HL, DH), jnp.float32),
            pltpu.VMEM((B, KV_USED, HL, DH), jnp.float32),
            pltpu.VMEM((B, SQ, DM), jnp.float32),
            pltpu.VMEM((N_DEV, B, BLK, DM), jnp.float32),
            pltpu.SemaphoreType.DMA((N_DEV,)),
            pltpu.SemaphoreType.DMA((N_DEV,)),
            pltpu.SemaphoreType.DMA((4,)),
            pltpu.SemaphoreType.DMA((N_DEV,)),
            pltpu.SemaphoreType.DMA((N_DEV,)),
            pltpu.SemaphoreType.DMA((N_DEV,)),
            pltpu.SemaphoreType.DMA((N_DEV,)),
        ],
        compiler_params=pltpu.CompilerParams(collective_id=0),
    )(x, Wq, K_ext, V_ext, Wo)


# baseline (device time: 286514 ns/iter reference)
import jax
import jax.numpy as jnp
from jax import lax
from jax.experimental import pallas as pl
from jax.experimental.pallas import tpu as pltpu

N_DEV = 4
B = 2
SQ = 512
SKV_LOC = 512
HL = 8
DH = 64
DM = 768
WIN = 128
KV_USED = SQ + WIN
KV1 = KV_USED - SKV_LOC
BLK = SQ // N_DEV
SCALE = 0.125
NEG = -1e9


def _mrc(src, dst, ssem, rsem, dev):
    return pltpu.make_async_remote_copy(
        src_ref=src, dst_ref=dst, send_sem=ssem, recv_sem=rsem,
        device_id=(dev,), device_id_type=pl.DeviceIdType.MESH,
    )


def kernel(x, Wq, K_ext, V_ext, Wo):
    def body(x_ref, wq_ref, k_ref, v_ref, wo_ref, out_ref,
             kbuf, vbuf, psum, arbuf,
             ksend, vsend, p1recv, rssend, rsrecv, agsend, agrecv):
        my = lax.axis_index("i")

        bar = pltpu.get_barrier_semaphore()
        for j in range(N_DEV):
            @pl.when(my != j)
            def _(j=j):
                pl.semaphore_signal(bar, inc=1, device_id=(j,),
                                    device_id_type=pl.DeviceIdType.MESH)
        pl.semaphore_wait(bar, N_DEV - 1)

        for s, (dst_off, n_rows) in ((0, (0, SKV_LOC)), (1, (SKV_LOC, KV1))):
            @pl.when(my == s)
            def _(s=s, dst_off=dst_off, n_rows=n_rows):
                descs = []
                for j in range(N_DEV):
                    if j == s:
                        continue
                    dk = _mrc(k_ref.at[:, pl.ds(0, n_rows), pl.ds(HL * j, HL), :],
                              kbuf.at[:, pl.ds(dst_off, n_rows)],
                              ksend.at[j], p1recv.at[2 * s], j)
                    dv = _mrc(v_ref.at[:, pl.ds(0, n_rows), pl.ds(HL * j, HL), :],
                              vbuf.at[:, pl.ds(dst_off, n_rows)],
                              vsend.at[j], p1recv.at[2 * s + 1], j)
                    dk.start()
                    dv.start()
                    descs.append((dk, dv))
                kbuf[:, dst_off:dst_off + n_rows] = \
                    k_ref[:, 0:n_rows, HL * s:HL * (s + 1), :]
                vbuf[:, dst_off:dst_off + n_rows] = \
                    v_ref[:, 0:n_rows, HL * s:HL * (s + 1), :]
                for dk, dv in descs:
                    dk.wait_send()
                    dv.wait_send()

        for s, (dst_off, n_rows) in ((0, (0, SKV_LOC)), (1, (SKV_LOC, KV1))):
            @pl.when(my != s)
            def _(s=s, dst_off=dst_off, n_rows=n_rows):
                rk = _mrc(kbuf.at[:, pl.ds(dst_off, n_rows)],
                          kbuf.at[:, pl.ds(dst_off, n_rows)],
                          ksend.at[s], p1recv.at[2 * s], my)
                rv = _mrc(vbuf.at[:, pl.ds(dst_off, n_rows)],
                          vbuf.at[:, pl.ds(dst_off, n_rows)],
                          vsend.at[s], p1recv.at[2 * s + 1], my)
                rk.wait_recv()
                rv.wait_recv()

        xq = lax.dot_general(
            jnp.reshape(x_ref[...], (B * SQ, DM)), wq_ref[...],
            (((1,), (0,)), ((), ())), preferred_element_type=jnp.float32)

        kv_k = kbuf[...]
        kv_v = vbuf[...]
        wo = wo_ref[...]

        qi = lax.broadcasted_iota(jnp.int32, (SQ, KV_USED), 0)
        ki = lax.broadcasted_iota(jnp.int32, (SQ, KV_USED), 1)
        mask = jnp.abs(qi - ki) <= WIN

        for b in range(B):
            ctxs = []
            for h in range(HL):
                q_bh = xq[b * SQ:(b + 1) * SQ, h * DH:(h + 1) * DH]
                k_bh = kv_k[b, :, h, :]
                v_bh = kv_v[b, :, h, :]
                sc = lax.dot_general(
                    q_bh, k_bh, (((1,), (1,)), ((), ())),
                    preferred_element_type=jnp.float32) * SCALE
                sc = jnp.where(mask, sc, NEG)
                m = jnp.max(sc, axis=-1, keepdims=True)
                w = jnp.exp(sc - m)
                w = w / jnp.sum(w, axis=-1, keepdims=True)
                ctxs.append(lax.dot_general(
                    w, v_bh, (((1,), (0,)), ((), ())),
                    preferred_element_type=jnp.float32))
            ctx_b = jnp.concatenate(ctxs, axis=1)
            psum[b] = lax.dot_general(
                ctx_b, wo, (((1,), (0,)), ((), ())),
                preferred_element_type=jnp.float32)

        for s in range(N_DEV):
            @pl.when(my == s)
            def _(s=s):
                for j in range(N_DEV):
                    if j == s:
                        continue
                    _mrc(psum.at[:, pl.ds(j * BLK, BLK), :], arbuf.at[s],
                         rssend.at[j], rsrecv.at[s], j).start()
                arbuf[s] = psum[:, s * BLK:(s + 1) * BLK, :]

        for s in range(N_DEV):
            @pl.when(my != s)
            def _(s=s):
                _mrc(arbuf.at[s], arbuf.at[s],
                     rssend.at[s], rsrecv.at[s], my).wait_recv()

        red = arbuf[0] + arbuf[1] + arbuf[2] + arbuf[3]

        for s in range(N_DEV):
            @pl.when(my == s)
            def _(s=s):
                out_ref[:, s * BLK:(s + 1) * BLK, :] = red
                for j in range(N_DEV):
                    if j == s:
                        continue
                    _mrc(out_ref.at[:, pl.ds(s * BLK, BLK), :],
                         out_ref.at[:, pl.ds(s * BLK, BLK), :],
                         agsend.at[j], agrecv.at[s], j).start()

        for s in range(N_DEV):
            @pl.when(my != s)
            def _(s=s):
                _mrc(out_ref.at[:, pl.ds(s * BLK, BLK), :],
                     out_ref.at[:, pl.ds(s * BLK, BLK), :],
                     agsend.at[s], agrecv.at[s], my).wait_recv()

        for s in range(N_DEV):
            @pl.when(my == s)
            def _(s=s):
                for j in range(N_DEV):
                    if j == s:
                        continue
                    _mrc(psum.at[:, pl.ds(j * BLK, BLK), :], arbuf.at[s],
                         rssend.at[j], rsrecv.at[s], j).wait_send()
                    _mrc(out_ref.at[:, pl.ds(s * BLK, BLK), :],
                         out_ref.at[:, pl.ds(s * BLK, BLK), :],
                         agsend.at[j], agrecv.at[s], j).wait_send()

    return pl.pallas_call(
        body,
        out_shape=jax.ShapeDtypeStruct((B, SQ, DM), jnp.float32),
        in_specs=[pl.BlockSpec(memory_space=pltpu.VMEM)] * 5,
        out_specs=pl.BlockSpec(memory_space=pltpu.VMEM),
        scratch_shapes=[
            pltpu.VMEM((B, KV_USED, HL, DH), jnp.float32),
            pltpu.VMEM((B, KV_USED, HL, DH), jnp.float32),
            pltpu.VMEM((B, SQ, DM), jnp.float32),
            pltpu.VMEM((N_DEV, B, BLK, DM), jnp.float32),
            pltpu.SemaphoreType.DMA((N_DEV,)),
            pltpu.SemaphoreType.DMA((N_DEV,)),
            pltpu.SemaphoreType.DMA((4,)),
            pltpu.SemaphoreType.DMA((N_DEV,)),
            pltpu.SemaphoreType.DMA((N_DEV,)),
            pltpu.SemaphoreType.DMA((N_DEV,)),
            pltpu.SemaphoreType.DMA((N_DEV,)),
        ],
        compiler_params=pltpu.CompilerParams(
            collective_id=0, vmem_limit_bytes=100 * 1024 * 1024),
    )(x, Wq, K_ext, V_ext, Wo)


# device time: 109652 ns/iter; 2.6129x vs baseline; 2.6129x over previous
import jax
import jax.numpy as jnp
from jax import lax
from jax.experimental import pallas as pl
from jax.experimental.pallas import tpu as pltpu

N_DEV = 4
B = 2
SQ = 512
SKV_LOC = 512
HQ = 32
HL = 8
DH = 64
HD = HL * DH
DM = 768
WIN = 128
KV_USED = SQ + WIN
KV1 = KV_USED - SKV_LOC
BLK = SQ // N_DEV
SCALE = 0.125
NEG = -1e9

_P1 = ((0, (0, SKV_LOC)), (1, (SKV_LOC, KV1)))


def _mrc(src, dst, ssem, rsem, dev):
    return pltpu.make_async_remote_copy(
        src_ref=src, dst_ref=dst, send_sem=ssem, recv_sem=rsem,
        device_id=(dev,), device_id_type=pl.DeviceIdType.MESH,
    )


def kernel(x, Wq, K_ext, V_ext, Wo):
    def body(x_ref, wq_ref, k_ref, v_ref, wo_ref, out_ref,
             kstage, vstage, kbuf, vbuf, psum, arbuf, agbuf,
             ksend, vsend, p1recv, rssend, rsrecv, agsend, agrecv):
        my = lax.axis_index("i")

        bar = pltpu.get_barrier_semaphore()
        for j in range(N_DEV):
            @pl.when(my != j)
            def _(j=j):
                pl.semaphore_signal(bar, inc=1, device_id=(j,),
                                    device_id_type=pl.DeviceIdType.MESH)
        pl.semaphore_wait(bar, N_DEV - 1)

        for s, (dst_off, n_rows) in _P1:
            @pl.when(my == s)
            def _(s=s, dst_off=dst_off, n_rows=n_rows):
                kstage[:, 0:n_rows] = k_ref[:, 0:n_rows].astype(jnp.bfloat16)
                vstage[:, 0:n_rows] = v_ref[:, 0:n_rows].astype(jnp.bfloat16)
                for j in range(N_DEV):
                    if j == s:
                        continue
                    _mrc(kstage.at[:, pl.ds(0, n_rows), pl.ds(HD * j, HD)],
                         kbuf.at[:, pl.ds(dst_off, n_rows)],
                         ksend.at[j], p1recv.at[2 * s], j).start()
                    _mrc(vstage.at[:, pl.ds(0, n_rows), pl.ds(HD * j, HD)],
                         vbuf.at[:, pl.ds(dst_off, n_rows)],
                         vsend.at[j], p1recv.at[2 * s + 1], j).start()
                kbuf[:, dst_off:dst_off + n_rows] = \
                    kstage[:, 0:n_rows, HD * s:HD * (s + 1)]
                vbuf[:, dst_off:dst_off + n_rows] = \
                    vstage[:, 0:n_rows, HD * s:HD * (s + 1)]

        xq = lax.dot_general(
            jnp.reshape(x_ref[...], (B * SQ, DM)).astype(jnp.bfloat16),
            wq_ref[...].astype(jnp.bfloat16),
            (((1,), (0,)), ((), ())),
            preferred_element_type=jnp.float32).astype(jnp.bfloat16)
        wo = wo_ref[...].astype(jnp.bfloat16)

        qi = lax.broadcasted_iota(jnp.int32, (SQ, KV_USED), 0)
        ki = lax.broadcasted_iota(jnp.int32, (SQ, KV_USED), 1)
        mask = jnp.abs(qi - ki) <= WIN

        for s, (dst_off, n_rows) in _P1:
            @pl.when(my != s)
            def _(s=s, dst_off=dst_off, n_rows=n_rows):
                _mrc(kbuf.at[:, pl.ds(dst_off, n_rows)],
                     kbuf.at[:, pl.ds(dst_off, n_rows)],
                     ksend.at[s], p1recv.at[2 * s], my).wait_recv()
                _mrc(vbuf.at[:, pl.ds(dst_off, n_rows)],
                     vbuf.at[:, pl.ds(dst_off, n_rows)],
                     vsend.at[s], p1recv.at[2 * s + 1], my).wait_recv()

        kv_k = kbuf[...]
        kv_v = vbuf[...]
        for b in range(B):
            ctxs = []
            for h in range(HL):
                q_bh = xq[b * SQ:(b + 1) * SQ, h * DH:(h + 1) * DH]
                k_bh = kv_k[b, :, h * DH:(h + 1) * DH]
                v_bh = kv_v[b, :, h * DH:(h + 1) * DH]
                sc = lax.dot_general(
                    q_bh, k_bh, (((1,), (1,)), ((), ())),
                    preferred_element_type=jnp.float32) * SCALE
                sc = jnp.where(mask, sc, NEG)
                m = jnp.max(sc, axis=-1, keepdims=True)
                w = jnp.exp(sc - m)
                l = jnp.sum(w, axis=-1, keepdims=True)
                ctx = lax.dot_general(
                    w.astype(jnp.bfloat16), v_bh, (((1,), (0,)), ((), ())),
                    preferred_element_type=jnp.float32)
                ctxs.append((ctx / l).astype(jnp.bfloat16))
            ctx_b = jnp.concatenate(ctxs, axis=1)
            psum[b] = lax.dot_general(
                ctx_b, wo, (((1,), (0,)), ((), ())),
                preferred_element_type=jnp.float32).astype(jnp.bfloat16)
            for s in range(N_DEV):
                @pl.when(my == s)
                def _(s=s, b=b):
                    for j in range(N_DEV):
                        if j == s:
                            continue
                        _mrc(psum.at[b, pl.ds(j * BLK, BLK), :],
                             arbuf.at[s, b],
                             rssend.at[j, b], rsrecv.at[s, b], j).start()
                    arbuf[s, b] = psum[b, s * BLK:(s + 1) * BLK, :]

        for s in range(N_DEV):
            @pl.when(my != s)
            def _(s=s):
                for b in range(B):
                    _mrc(arbuf.at[s, b], arbuf.at[s, b],
                         rssend.at[s, b], rsrecv.at[s, b], my).wait_recv()

        red = (arbuf[0].astype(jnp.float32) + arbuf[1].astype(jnp.float32) +
               arbuf[2].astype(jnp.float32) + arbuf[3].astype(jnp.float32))

        for s in range(N_DEV):
            @pl.when(my == s)
            def _(s=s):
                agbuf[s] = red.astype(jnp.bfloat16)
                for j in range(N_DEV):
                    if j == s:
                        continue
                    _mrc(agbuf.at[s], agbuf.at[s],
                         agsend.at[j], agrecv.at[s], j).start()

        for s in range(N_DEV):
            @pl.when(my != s)
            def _(s=s):
                _mrc(agbuf.at[s], agbuf.at[s],
                     agsend.at[s], agrecv.at[s], my).wait_recv()

        for s in range(N_DEV):
            out_ref[:, s * BLK:(s + 1) * BLK, :] = agbuf[s].astype(jnp.float32)

        for s, (dst_off, n_rows) in _P1:
            @pl.when(my == s)
            def _(s=s, dst_off=dst_off, n_rows=n_rows):
                for j in range(N_DEV):
                    if j == s:
                        continue
                    _mrc(kstage.at[:, pl.ds(0, n_rows), pl.ds(HD * j, HD)],
                         kbuf.at[:, pl.ds(dst_off, n_rows)],
                         ksend.at[j], p1recv.at[2 * s], j).wait_send()
                    _mrc(vstage.at[:, pl.ds(0, n_rows), pl.ds(HD * j, HD)],
                         vbuf.at[:, pl.ds(dst_off, n_rows)],
                         vsend.at[j], p1recv.at[2 * s + 1], j).wait_send()
        for s in range(N_DEV):
            @pl.when(my == s)
            def _(s=s):
                for j in range(N_DEV):
                    if j == s:
                        continue
                    for b in range(B):
                        _mrc(psum.at[b, pl.ds(j * BLK, BLK), :],
                             arbuf.at[s, b],
                             rssend.at[j, b], rsrecv.at[s, b], j).wait_send()
                    _mrc(agbuf.at[s], agbuf.at[s],
                         agsend.at[j], agrecv.at[s], j).wait_send()

    return pl.pallas_call(
        body,
        out_shape=jax.ShapeDtypeStruct((B, SQ, DM), jnp.float32),
        in_specs=[pl.BlockSpec(memory_space=pltpu.VMEM)] * 5,
        out_specs=pl.BlockSpec(memory_space=pltpu.VMEM),
        scratch_shapes=[
            pltpu.VMEM((B, SKV_LOC, HQ * DH), jnp.bfloat16),
            pltpu.VMEM((B, SKV_LOC, HQ * DH), jnp.bfloat16),
            pltpu.VMEM((B, KV_USED, HD), jnp.bfloat16),
            pltpu.VMEM((B, KV_USED, HD), jnp.bfloat16),
            pltpu.VMEM((B, SQ, DM), jnp.bfloat16),
            pltpu.VMEM((N_DEV, B, BLK, DM), jnp.bfloat16),
            pltpu.VMEM((N_DEV, B, BLK, DM), jnp.bfloat16),
            pltpu.SemaphoreType.DMA((N_DEV,)),
            pltpu.SemaphoreType.DMA((N_DEV,)),
            pltpu.SemaphoreType.DMA((4,)),
            pltpu.SemaphoreType.DMA((N_DEV, B)),
            pltpu.SemaphoreType.DMA((N_DEV, B)),
            pltpu.SemaphoreType.DMA((N_DEV,)),
            pltpu.SemaphoreType.DMA((N_DEV,)),
        ],
        compiler_params=pltpu.CompilerParams(
            collective_id=0, vmem_limit_bytes=100 * 1024 * 1024),
    )(x, Wq,
      jnp.reshape(K_ext, (B, SKV_LOC, HQ * DH)),
      jnp.reshape(V_ext, (B, SKV_LOC, HQ * DH)),
      Wo)


# device time: 104073 ns/iter; 2.7530x vs baseline; 1.0536x over previous
import jax
import jax.numpy as jnp
from jax import lax
from jax.experimental import pallas as pl
from jax.experimental.pallas import tpu as pltpu

N_DEV = 4
B = 2
SQ = 512
SKV_LOC = 512
HQ = 32
HL = 8
DH = 64
HD = HL * DH
DM = 768
WIN = 128
KV_USED = SQ + WIN
KV1 = KV_USED - SKV_LOC
BLK = SQ // N_DEV
SCALE = 0.125
NEG = -1e9

_P1 = ((0, (0, SKV_LOC)), (1, (SKV_LOC, KV1)))


def _mrc(src, dst, ssem, rsem, dev):
    return pltpu.make_async_remote_copy(
        src_ref=src, dst_ref=dst, send_sem=ssem, recv_sem=rsem,
        device_id=(dev,), device_id_type=pl.DeviceIdType.MESH,
    )


def kernel(x, Wq, K_ext, V_ext, Wo):
    def body(x_ref, wq_ref, k_ref, v_ref, wo_ref, out_ref,
             kstage, vstage, kbuf, vbuf, psum, arbuf, agbuf,
             ksend, vsend, p1recv, rssend, rsrecv, agsend, agrecv):
        my = lax.axis_index("i")

        bar = pltpu.get_barrier_semaphore()
        for j in range(N_DEV):
            @pl.when(my != j)
            def _(j=j):
                pl.semaphore_signal(bar, inc=1, device_id=(j,),
                                    device_id_type=pl.DeviceIdType.MESH)
        pl.semaphore_wait(bar, N_DEV - 1)

        for s, (dst_off, n_rows) in _P1:
            @pl.when(my == s)
            def _(s=s, dst_off=dst_off, n_rows=n_rows):
                for j in range(N_DEV):
                    if j == s:
                        continue
                    kstage[:, 0:n_rows, HD * j:HD * (j + 1)] = \
                        k_ref[:, 0:n_rows, HD * j:HD * (j + 1)].astype(jnp.bfloat16)
                    _mrc(kstage.at[:, pl.ds(0, n_rows), pl.ds(HD * j, HD)],
                         kbuf.at[:, pl.ds(dst_off, n_rows)],
                         ksend.at[j], p1recv.at[2 * s], j).start()
                    vstage[:, 0:n_rows, HD * j:HD * (j + 1)] = \
                        v_ref[:, 0:n_rows, HD * j:HD * (j + 1)].astype(jnp.bfloat16)
                    _mrc(vstage.at[:, pl.ds(0, n_rows), pl.ds(HD * j, HD)],
                         vbuf.at[:, pl.ds(dst_off, n_rows)],
                         vsend.at[j], p1recv.at[2 * s + 1], j).start()
                kbuf[:, dst_off:dst_off + n_rows] = \
                    k_ref[:, 0:n_rows, HD * s:HD * (s + 1)].astype(jnp.bfloat16)
                vbuf[:, dst_off:dst_off + n_rows] = \
                    v_ref[:, 0:n_rows, HD * s:HD * (s + 1)].astype(jnp.bfloat16)

        xq = lax.dot_general(
            jnp.reshape(x_ref[...], (B * SQ, DM)).astype(jnp.bfloat16),
            wq_ref[...].astype(jnp.bfloat16),
            (((1,), (0,)), ((), ())),
            preferred_element_type=jnp.float32).astype(jnp.bfloat16)
        wo = wo_ref[...].astype(jnp.bfloat16)

        qi = lax.broadcasted_iota(jnp.int32, (SQ, KV_USED), 0)
        ki = lax.broadcasted_iota(jnp.int32, (SQ, KV_USED), 1)
        maskf = (jnp.abs(qi - ki) <= WIN).astype(jnp.float32)

        for s, (dst_off, n_rows) in _P1:
            @pl.when(my != s)
            def _(s=s, dst_off=dst_off, n_rows=n_rows):
                _mrc(kbuf.at[:, pl.ds(dst_off, n_rows)],
                     kbuf.at[:, pl.ds(dst_off, n_rows)],
                     ksend.at[s], p1recv.at[2 * s], my).wait_recv()
                _mrc(vbuf.at[:, pl.ds(dst_off, n_rows)],
                     vbuf.at[:, pl.ds(dst_off, n_rows)],
                     vsend.at[s], p1recv.at[2 * s + 1], my).wait_recv()

        kv_k = kbuf[...]
        kv_v = vbuf[...]
        for b in range(B):
            ctxs = []
            for h in range(HL):
                q_bh = xq[b * SQ:(b + 1) * SQ, h * DH:(h + 1) * DH]
                k_bh = kv_k[b, :, h * DH:(h + 1) * DH]
                v_bh = kv_v[b, :, h * DH:(h + 1) * DH]
                sc = lax.dot_general(
                    q_bh, k_bh, (((1,), (1,)), ((), ())),
                    preferred_element_type=jnp.float32)
                w = jnp.exp(sc * SCALE) * maskf
                l = jnp.sum(w, axis=-1, keepdims=True)
                ctx = lax.dot_general(
                    w.astype(jnp.bfloat16), v_bh, (((1,), (0,)), ((), ())),
                    preferred_element_type=jnp.float32)
                ctxs.append((ctx / l).astype(jnp.bfloat16))
            ctx_b = jnp.concatenate(ctxs, axis=1)
            psum[b] = lax.dot_general(
                ctx_b, wo, (((1,), (0,)), ((), ())),
                preferred_element_type=jnp.float32).astype(jnp.bfloat16)
            for s in range(N_DEV):
                @pl.when(my == s)
                def _(s=s, b=b):
                    for j in range(N_DEV):
                        if j == s:
                            continue
                        _mrc(psum.at[b, pl.ds(j * BLK, BLK), :],
                             arbuf.at[s, b],
                             rssend.at[j, b], rsrecv.at[s, b], j).start()
                    arbuf[s, b] = psum[b, s * BLK:(s + 1) * BLK, :]

        for s in range(N_DEV):
            @pl.when(my != s)
            def _(s=s):
                for b in range(B):
                    _mrc(arbuf.at[s, b], arbuf.at[s, b],
                         rssend.at[s, b], rsrecv.at[s, b], my).wait_recv()

        red = (arbuf[0].astype(jnp.float32) + arbuf[1].astype(jnp.float32) +
               arbuf[2].astype(jnp.float32) + arbuf[3].astype(jnp.float32))

        for s in range(N_DEV):
            @pl.when(my == s)
            def _(s=s):
                agbuf[s] = red.astype(jnp.bfloat16)
                for j in range(N_DEV):
                    if j == s:
                        continue
                    _mrc(agbuf.at[s], agbuf.at[s],
                         agsend.at[j], agrecv.at[s], j).start()

        for s in range(N_DEV):
            @pl.when(my != s)
            def _(s=s):
                _mrc(agbuf.at[s], agbuf.at[s],
                     agsend.at[s], agrecv.at[s], my).wait_recv()

        for s in range(N_DEV):
            out_ref[:, s * BLK:(s + 1) * BLK, :] = agbuf[s].astype(jnp.float32)

        for s, (dst_off, n_rows) in _P1:
            @pl.when(my == s)
            def _(s=s, dst_off=dst_off, n_rows=n_rows):
                for j in range(N_DEV):
                    if j == s:
                        continue
                    _mrc(kstage.at[:, pl.ds(0, n_rows), pl.ds(HD * j, HD)],
                         kbuf.at[:, pl.ds(dst_off, n_rows)],
                         ksend.at[j], p1recv.at[2 * s], j).wait_send()
                    _mrc(vstage.at[:, pl.ds(0, n_rows), pl.ds(HD * j, HD)],
                         vbuf.at[:, pl.ds(dst_off, n_rows)],
                         vsend.at[j], p1recv.at[2 * s + 1], j).wait_send()
        for s in range(N_DEV):
            @pl.when(my == s)
            def _(s=s):
                for j in range(N_DEV):
                    if j == s:
                        continue
                    for b in range(B):
                        _mrc(psum.at[b, pl.ds(j * BLK, BLK), :],
                             arbuf.at[s, b],
                             rssend.at[j, b], rsrecv.at[s, b], j).wait_send()
                    _mrc(agbuf.at[s], agbuf.at[s],
                         agsend.at[j], agrecv.at[s], j).wait_send()

    return pl.pallas_call(
        body,
        out_shape=jax.ShapeDtypeStruct((B, SQ, DM), jnp.float32),
        in_specs=[pl.BlockSpec(memory_space=pltpu.VMEM)] * 5,
        out_specs=pl.BlockSpec(memory_space=pltpu.VMEM),
        scratch_shapes=[
            pltpu.VMEM((B, SKV_LOC, HQ * DH), jnp.bfloat16),
            pltpu.VMEM((B, SKV_LOC, HQ * DH), jnp.bfloat16),
            pltpu.VMEM((B, KV_USED, HD), jnp.bfloat16),
            pltpu.VMEM((B, KV_USED, HD), jnp.bfloat16),
            pltpu.VMEM((B, SQ, DM), jnp.bfloat16),
            pltpu.VMEM((N_DEV, B, BLK, DM), jnp.bfloat16),
            pltpu.VMEM((N_DEV, B, BLK, DM), jnp.bfloat16),
            pltpu.SemaphoreType.DMA((N_DEV,)),
            pltpu.SemaphoreType.DMA((N_DEV,)),
            pltpu.SemaphoreType.DMA((4,)),
            pltpu.SemaphoreType.DMA((N_DEV, B)),
            pltpu.SemaphoreType.DMA((N_DEV, B)),
            pltpu.SemaphoreType.DMA((N_DEV,)),
            pltpu.SemaphoreType.DMA((N_DEV,)),
        ],
        compiler_params=pltpu.CompilerParams(
            collective_id=0, vmem_limit_bytes=100 * 1024 * 1024),
    )(x, Wq,
      jnp.reshape(K_ext, (B, SKV_LOC, HQ * DH)),
      jnp.reshape(V_ext, (B, SKV_LOC, HQ * DH)),
      Wo)


# device time: 98118 ns/iter; 2.9201x vs baseline; 1.0607x over previous
import jax
import jax.numpy as jnp
from jax import lax
from jax.experimental import pallas as pl
from jax.experimental.pallas import tpu as pltpu

N_DEV = 4
B = 2
SQ = 512
SKV_LOC = 512
HQ = 32
HL = 8
DH = 64
HD = HL * DH
DM = 768
WIN = 128
KV_USED = SQ + WIN
CH = 256
KV1 = KV_USED - SKV_LOC
BLK = SQ // N_DEV
SCALE = 0.125

_CHUNK_A = (0, CH, 0, 0, 0)
_CHUNK_B = (CH, CH, 0, CH, 2)
_CHUNK_C = (2 * CH, KV1, 1, 0, 4)


def _mrc(src, dst, ssem, rsem, dev):
    return pltpu.make_async_remote_copy(
        src_ref=src, dst_ref=dst, send_sem=ssem, recv_sem=rsem,
        device_id=(dev,), device_id_type=pl.DeviceIdType.MESH,
    )


def kernel(x, Wq, K_ext, V_ext, Wo):
    def body(x_ref, wq_ref, k_ref, v_ref, wo_ref, out_ref,
             kbuf, vbuf, psum, arbuf, agbuf,
             ksend, vsend, p1recv, rssend, rsrecv, agsend, agrecv):
        my = lax.axis_index("i")

        def chunk_sends(dst_off, n_rows, s, src_off, sem, cslot):
            descs = []
            for j in range(N_DEV):
                if j == s:
                    continue
                dk = _mrc(k_ref.at[:, pl.ds(src_off, n_rows), pl.ds(HD * j, HD)],
                          kbuf.at[:, pl.ds(dst_off, n_rows)],
                          ksend.at[j, cslot], p1recv.at[sem], j)
                dv = _mrc(v_ref.at[:, pl.ds(src_off, n_rows), pl.ds(HD * j, HD)],
                          vbuf.at[:, pl.ds(dst_off, n_rows)],
                          vsend.at[j, cslot], p1recv.at[sem + 1], j)
                descs.append((dk, dv))
            return descs

        def chunk_wait(dst_off, n_rows, sem):
            _mrc(kbuf.at[:, pl.ds(dst_off, n_rows)],
                 kbuf.at[:, pl.ds(dst_off, n_rows)],
                 ksend.at[0, 0], p1recv.at[sem], my).wait_recv()
            _mrc(vbuf.at[:, pl.ds(dst_off, n_rows)],
                 vbuf.at[:, pl.ds(dst_off, n_rows)],
                 vsend.at[0, 0], p1recv.at[sem + 1], my).wait_recv()

        bar = pltpu.get_barrier_semaphore()
        for j in range(N_DEV):
            @pl.when(my != j)
            def _(j=j):
                pl.semaphore_signal(bar, inc=1, device_id=(j,),
                                    device_id_type=pl.DeviceIdType.MESH)
        pl.semaphore_wait(bar, N_DEV - 1)

        @pl.when(my == 0)
        def _():
            for dk, dv in chunk_sends(0, CH, 0, 0, 0, 0):
                dk.start()
                dv.start()
            kbuf[:, 0:SKV_LOC] = k_ref[:, :, 0:HD]
            vbuf[:, 0:SKV_LOC] = v_ref[:, :, 0:HD]

        @pl.when(my == 1)
        def _():
            for dk, dv in chunk_sends(2 * CH, KV1, 1, 0, 4, 0):
                dk.start()
                dv.start()
            kbuf[:, 2 * CH:KV_USED] = k_ref[:, 0:KV1, HD:2 * HD]
            vbuf[:, 2 * CH:KV_USED] = v_ref[:, 0:KV1, HD:2 * HD]

        xq = (lax.dot_general(
            jnp.reshape(x_ref[...], (B * SQ, DM)).astype(jnp.bfloat16),
            wq_ref[...].astype(jnp.bfloat16),
            (((1,), (0,)), ((), ())),
            preferred_element_type=jnp.float32) * SCALE).astype(jnp.bfloat16)
        wo = wo_ref[...].astype(jnp.bfloat16)

        qi = lax.broadcasted_iota(jnp.int32, (SQ, KV_USED), 0)
        ki = lax.broadcasted_iota(jnp.int32, (SQ, KV_USED), 1)
        maskb = (jnp.abs(qi - ki) <= WIN).astype(jnp.bfloat16)

        @pl.when(my == 0)
        def _():
            for dk, dv in chunk_sends(0, CH, 0, 0, 0, 0):
                dk.wait_send()
                dv.wait_send()
            for dk, dv in chunk_sends(CH, CH, 0, CH, 2, 1):
                dk.start()
                dv.start()

        def chunk_attn(b, h, dst_off, n_rows):
            q_bh = xq[b * SQ:(b + 1) * SQ, h * DH:(h + 1) * DH]
            sc = lax.dot_general(
                q_bh, kbuf[b, dst_off:dst_off + n_rows, h * DH:(h + 1) * DH],
                (((1,), (1,)), ((), ())),
                preferred_element_type=jnp.float32).astype(jnp.bfloat16)
            w = jnp.exp(sc) * maskb[:, dst_off:dst_off + n_rows]
            l = jnp.sum(w.astype(jnp.float32), axis=-1, keepdims=True)
            ctx = lax.dot_general(
                w, vbuf[b, dst_off:dst_off + n_rows, h * DH:(h + 1) * DH],
                (((1,), (0,)), ((), ())),
                preferred_element_type=jnp.float32)
            return ctx, l

        parts = {}
        for dst_off, n_rows, sender, _src, sem in (_CHUNK_C, _CHUNK_A):
            @pl.when(my != sender)
            def _(dst_off=dst_off, n_rows=n_rows, sem=sem):
                chunk_wait(dst_off, n_rows, sem)
            for b in range(B):
                for h in range(HL):
                    ctx, l = chunk_attn(b, h, dst_off, n_rows)
                    if (b, h) in parts:
                        pc, plsum = parts[(b, h)]
                        parts[(b, h)] = (pc + ctx, plsum + l)
                    else:
                        parts[(b, h)] = (ctx, l)

        @pl.when(my != 0)
        def _():
            chunk_wait(CH, CH, 2)

        for b in range(B):
            ctxs = []
            for h in range(HL):
                ctx, l = chunk_attn(b, h, CH, CH)
                pc, plsum = parts[(b, h)]
                ctxs.append(((pc + ctx) / (plsum + l)).astype(jnp.bfloat16))
            ctx_b = jnp.concatenate(ctxs, axis=1)
            psum[b] = lax.dot_general(
                ctx_b, wo, (((1,), (0,)), ((), ())),
                preferred_element_type=jnp.float32).astype(jnp.bfloat16)
            for s in range(N_DEV):
                @pl.when(my == s)
                def _(s=s, b=b):
                    for j in range(N_DEV):
                        if j == s:
                            continue
                        _mrc(psum.at[b, pl.ds(j * BLK, BLK), :],
                             arbuf.at[s, b],
                             rssend.at[j, b], rsrecv.at[s, b], j).start()
                    arbuf[s, b] = psum[b, s * BLK:(s + 1) * BLK, :]

        for s in range(N_DEV):
            @pl.when(my != s)
            def _(s=s):
                for b in range(B):
                    _mrc(arbuf.at[s, b], arbuf.at[s, b],
                         rssend.at[s, b], rsrecv.at[s, b], my).wait_recv()

        red = (arbuf[0].astype(jnp.float32) + arbuf[1].astype(jnp.float32) +
               arbuf[2].astype(jnp.float32) + arbuf[3].astype(jnp.float32))

        for s in range(N_DEV):
            @pl.when(my == s)
            def _(s=s):
                agbuf[s] = red.astype(jnp.bfloat16)
                out_ref[:, s * BLK:(s + 1) * BLK, :] = red
                for j in range(N_DEV):
                    if j == s:
                        continue
                    _mrc(agbuf.at[s], agbuf.at[s],
                         agsend.at[j], agrecv.at[s], j).start()

        for s in range(N_DEV):
            @pl.when(my != s)
            def _(s=s):
                _mrc(agbuf.at[s], agbuf.at[s],
                     agsend.at[s], agrecv.at[s], my).wait_recv()
                out_ref[:, s * BLK:(s + 1) * BLK, :] = \
                    agbuf[s].astype(jnp.float32)

        @pl.when(my == 0)
        def _():
            for dk, dv in chunk_sends(CH, CH, 0, CH, 2, 1):
                dk.wait_send()
                dv.wait_send()

        @pl.when(my == 1)
        def _():
            for dk, dv in chunk_sends(2 * CH, KV1, 1, 0, 4, 0):
                dk.wait_send()
                dv.wait_send()

        for s in range(N_DEV):
            @pl.when(my == s)
            def _(s=s):
                for j in range(N_DEV):
                    if j == s:
                        continue
                    for b in range(B):
                        _mrc(psum.at[b, pl.ds(j * BLK, BLK), :],
                             arbuf.at[s, b],
                             rssend.at[j, b], rsrecv.at[s, b], j).wait_send()
                    _mrc(agbuf.at[s], agbuf.at[s],
                         agsend.at[j], agrecv.at[s], j).wait_send()

    return pl.pallas_call(
        body,
        out_shape=jax.ShapeDtypeStruct((B, SQ, DM), jnp.float32),
        in_specs=[pl.BlockSpec(memory_space=pltpu.VMEM)] * 5,
        out_specs=pl.BlockSpec(memory_space=pltpu.VMEM),
        scratch_shapes=[
            pltpu.VMEM((B, KV_USED, HD), jnp.bfloat16),
            pltpu.VMEM((B, KV_USED, HD), jnp.bfloat16),
            pltpu.VMEM((B, SQ, DM), jnp.bfloat16),
            pltpu.VMEM((N_DEV, B, BLK, DM), jnp.bfloat16),
            pltpu.VMEM((N_DEV, B, BLK, DM), jnp.bfloat16),
            pltpu.SemaphoreType.DMA((N_DEV, 2)),
            pltpu.SemaphoreType.DMA((N_DEV, 2)),
            pltpu.SemaphoreType.DMA((6,)),
            pltpu.SemaphoreType.DMA((N_DEV, B)),
            pltpu.SemaphoreType.DMA((N_DEV, B)),
            pltpu.SemaphoreType.DMA((N_DEV,)),
            pltpu.SemaphoreType.DMA((N_DEV,)),
        ],
        compiler_params=pltpu.CompilerParams(
            collective_id=0, vmem_limit_bytes=100 * 1024 * 1024),
    )(x, Wq,
      jnp.reshape(K_ext.astype(jnp.bfloat16), (B, SKV_LOC, HQ * DH)),
      jnp.reshape(V_ext.astype(jnp.bfloat16), (B, SKV_LOC, HQ * DH)),
      Wo)
